# baseline (speedup 1.0000x reference)
import functools

import jax
import jax.numpy as jnp
from jax import lax
from jax.experimental import pallas as pl
from jax.experimental.pallas import tpu as pltpu
from jax.experimental.pallas import tpu_sc as plsc

N = 10000
S = 32
D = 128
LANES = 16
NVEC = D // LANES

NC = 2
NS = 16
NW = NC * NS
C = 4
CS = C * S
NP = 10240
TOTCH = NP // C
NBUF = 4
KF = 132
KS = 28
assert NS * (KF + KS) == TOTCH and KF % NBUF == 0 and KS % NBUF == 0
IDXPAD = TOTCH + KF

BLK = 1000


_sc_mesh = plsc.VectorSubcoreMesh(
    core_axis_name="c", subcore_axis_name="s", num_cores=NC, num_subcores=NS
)

DW = D // 2


@functools.partial(
    pl.kernel,
    out_type=jax.ShapeDtypeStruct((NP, DW), jnp.int32),
    mesh=_sc_mesh,
    scratch_types=[
        pltpu.VMEM((KF, CS), jnp.int32),
        pltpu.VMEM((NBUF, CS, DW), jnp.int32),
        pltpu.VMEM((KF * C, DW), jnp.int32),
    ]
    + [pltpu.SemaphoreType.DMA] * NBUF,
    compiler_params=pltpu.CompilerParams(
        use_tc_tiling_on_sc=False, needs_layout_passes=False
    ),
)
def _sc_gather_max(h_hbm, idx_hbm, out_hbm, idx_v, rows_v, out_v, *sems):
    core = lax.axis_index("c")
    sub = lax.axis_index("s")
    on_fast = core == 0
    start = lax.select(on_fast, sub * KF, NS * KF + sub * KS)
    kc = lax.select(on_fast, KF, KS)

    @pl.when(kc > 0)
    def _():
        pltpu.sync_copy(idx_hbm.at[pl.ds(start, KF)], idx_v)
        for b in range(NBUF):
            pltpu.async_copy(h_hbm.at[idx_v.at[b]], rows_v.at[b], sems[b])

    nvec = DW // LANES

    def jbody(j, carry):
        for b in range(NBUF):
            chunk = j * NBUF + b
            pltpu.make_async_copy(
                h_hbm.at[idx_v.at[0]], rows_v.at[b], sems[b]
            ).wait()
            rv = rows_v.at[b]
            for c in range(C):
                accs = [
                    plsc.bitcast(rv[c * S, pl.ds(d * LANES, LANES)], jnp.bfloat16)
                    for d in range(nvec)
                ]
                for s in range(1, S):
                    r = c * S + s
                    accs = [
                        jnp.maximum(
                            a,
                            plsc.bitcast(
                                rv[r, pl.ds(d * LANES, LANES)], jnp.bfloat16
                            ),
                        )
                        for d, a in enumerate(accs)
                    ]
                node = chunk * C + c
                for d in range(nvec):
                    out_v[node, pl.ds(d * LANES, LANES)] = plsc.bitcast(
                        accs[d], jnp.int32
                    )

            @pl.when(chunk + NBUF < kc)
            def _():
                pltpu.async_copy(
                    h_hbm.at[idx_v.at[chunk + NBUF]], rows_v.at[b], sems[b]
                )

        return carry

    lax.fori_loop(0, kc // NBUF, jbody, 0)

    @pl.when(on_fast)
    def _():
        pltpu.sync_copy(out_v, out_hbm.at[pl.ds(start * C, KF * C)])

    if KS > 0:

        @pl.when(jnp.logical_not(on_fast))
        def _():
            pltpu.sync_copy(
                out_v.at[pl.ds(0, KS * C)], out_hbm.at[pl.ds(start * C, KS * C)]
            )


def _pack_pairs(y):
    yb = y.astype(jnp.bfloat16)
    lo = lax.bitcast_convert_type(yb[:, :DW], jnp.uint16).astype(jnp.int32)
    hi = lax.bitcast_convert_type(yb[:, DW:], jnp.uint16).astype(jnp.int32)
    return lo | (hi << 16)


def _unpack_pairs(w):
    al = lax.bitcast_convert_type(w << 16, jnp.float32)
    ah = lax.bitcast_convert_type(w & jnp.int32(-65536), jnp.float32)
    return al, ah


def _k_mm_relu(x_ref, w_ref, b_ref, o_ref):
    o_ref[...] = _pack_pairs(
        jnp.maximum(
            jnp.dot(x_ref[...], w_ref[...], preferred_element_type=jnp.float32)
            + b_ref[...],
            0.0,
        )
    )


def _mm_relu(x, w, b):
    return pl.pallas_call(
        _k_mm_relu,
        grid=(N // BLK,),
        in_specs=[
            pl.BlockSpec((BLK, D), lambda i: (i, 0)),
            pl.BlockSpec((D, D), lambda i: (0, 0)),
            pl.BlockSpec((1, D), lambda i: (0, 0)),
        ],
        out_specs=pl.BlockSpec((BLK, DW), lambda i: (i, 0)),
        out_shape=jax.ShapeDtypeStruct((N, DW), jnp.int32),
    )(x, w, b.reshape(1, D))


def _k_fc_sums(x_ref, a_ref, wa_ref, wb_ref, b_ref, y_ref, s_ref):
    y = jnp.dot(x_ref[...], wa_ref[...], preferred_element_type=jnp.float32)
    al, ah = _unpack_pairs(a_ref[...])
    wb = wb_ref[...]
    y = y + jnp.dot(al, wb[:DW], preferred_element_type=jnp.float32)
    y = y + jnp.dot(ah, wb[DW:], preferred_element_type=jnp.float32)
    y = jnp.maximum(y + b_ref[...], 0.0)
    y_ref[...] = y

    @pl.when(pl.program_id(0) == 0)
    def _():
        s_ref[...] = jnp.zeros_like(s_ref)

    s_ref[...] += jnp.concatenate(
        [
            jnp.sum(y, axis=0, keepdims=True),
            jnp.sum(y * y, axis=0, keepdims=True),
            jnp.zeros((6, D), jnp.float32),
        ],
        axis=0,
    )


def _fc_sums(x, a, wa, wb, b):
    return pl.pallas_call(
        _k_fc_sums,
        grid=(N // BLK,),
        in_specs=[
            pl.BlockSpec((BLK, D), lambda i: (i, 0)),
            pl.BlockSpec((BLK, DW), lambda i: (i, 0)),
            pl.BlockSpec((D, D), lambda i: (0, 0)),
            pl.BlockSpec((D, D), lambda i: (0, 0)),
            pl.BlockSpec((1, D), lambda i: (0, 0)),
        ],
        out_specs=[
            pl.BlockSpec((BLK, D), lambda i: (i, 0)),
            pl.BlockSpec((8, D), lambda i: (0, 0)),
        ],
        out_shape=[
            jax.ShapeDtypeStruct((N, D), jnp.float32),
            jax.ShapeDtypeStruct((8, D), jnp.float32),
        ],
    )(x, a, wa, wb, b.reshape(1, D))


def _k_bn_norm_mm(y_ref, s_ref, g_ref, bt_ref, w2_ref, b2_ref, o1_ref, h2_ref):
    s = s_ref[...]
    mean = s[0:1, :] * (1.0 / N)
    var = s[1:2, :] * (1.0 / N) - mean * mean
    inv = lax.rsqrt(var + 1e-5)
    o = g_ref[...] * (y_ref[...] - mean) * inv + bt_ref[...]
    nrm = jnp.sqrt(jnp.sum(o * o, axis=1, keepdims=True))
    o = o / (nrm + 1e-6)
    o1_ref[...] = o
    h2_ref[...] = _pack_pairs(
        jnp.maximum(
            jnp.dot(o, w2_ref[...], preferred_element_type=jnp.float32)
            + b2_ref[...],
            0.0,
        )
    )


def _bn_norm_mm(y, sums, g, bt, w2, b2):
    return pl.pallas_call(
        _k_bn_norm_mm,
        grid=(N // BLK,),
        in_specs=[
            pl.BlockSpec((BLK, D), lambda i: (i, 0)),
            pl.BlockSpec((8, D), lambda i: (0, 0)),
            pl.BlockSpec((1, D), lambda i: (0, 0)),
            pl.BlockSpec((1, D), lambda i: (0, 0)),
            pl.BlockSpec((D, D), lambda i: (0, 0)),
            pl.BlockSpec((1, D), lambda i: (0, 0)),
        ],
        out_specs=[
            pl.BlockSpec((BLK, D), lambda i: (i, 0)),
            pl.BlockSpec((BLK, DW), lambda i: (i, 0)),
        ],
        out_shape=[
            jax.ShapeDtypeStruct((N, D), jnp.float32),
            jax.ShapeDtypeStruct((N, DW), jnp.int32),
        ],
    )(y, sums, g.reshape(1, D), bt.reshape(1, D), w2, b2.reshape(1, D))


def _k_fc_out(x_ref, a_ref, wa_ref, wb_ref, b_ref, o_ref):
    o = jnp.dot(x_ref[...], wa_ref[...], preferred_element_type=jnp.float32)
    al, ah = _unpack_pairs(a_ref[...])
    wb = wb_ref[...]
    o = o + jnp.dot(al, wb[:DW], preferred_element_type=jnp.float32)
    o = o + jnp.dot(ah, wb[DW:], preferred_element_type=jnp.float32)
    o_ref[...] = o + b_ref[...]


def _fc_out(x, a, wa, wb, b):
    return pl.pallas_call(
        _k_fc_out,
        grid=(N // BLK,),
        in_specs=[
            pl.BlockSpec((BLK, D), lambda i: (i, 0)),
            pl.BlockSpec((BLK, DW), lambda i: (i, 0)),
            pl.BlockSpec((D, D), lambda i: (0, 0)),
            pl.BlockSpec((D, D), lambda i: (0, 0)),
            pl.BlockSpec((1, D), lambda i: (0, 0)),
        ],
        out_specs=pl.BlockSpec((BLK, D), lambda i: (i, 0)),
        out_shape=jax.ShapeDtypeStruct((N, D), jnp.float32),
    )(x, a, wa, wb, b.reshape(1, D))


def kernel(features, neigh, Wpool1, bpool1, Wfc1, bfc1, gamma1, beta1,
           Wpool2, bpool2, Wfc2, bfc2):
    x = features.astype(jnp.float32)
    idx = jnp.concatenate(
        [neigh.astype(jnp.int32), jnp.zeros((IDXPAD * C - N, S), jnp.int32)],
        axis=0,
    ).reshape(IDXPAD, CS)

    h1 = _mm_relu(x, Wpool1, bpool1)
    agg1 = _sc_gather_max(h1, idx)[:N]
    y, sums = _fc_sums(x, agg1, Wfc1[:D], Wfc1[D:], bfc1)
    out1, h2 = _bn_norm_mm(y, sums, gamma1, beta1, Wpool2, bpool2)
    agg2 = _sc_gather_max(h2, idx)[:N]
    return _fc_out(out1, agg2, Wfc2[:D], Wfc2[D:], bfc2)


# speedup vs baseline: 1.0165x; 1.0165x over previous
import functools

import jax
import jax.numpy as jnp
from jax import lax
from jax.experimental import pallas as pl
from jax.experimental.pallas import tpu as pltpu
from jax.experimental.pallas import tpu_sc as plsc

N = 10000
S = 32
D = 128
LANES = 16
NVEC = D // LANES

NC = 2
NS = 16
NW = NC * NS
C = 8
CS = C * S
NP = 10240
TOTCH = NP // C
NBUF = 2
KF = 60
KS = 20
assert NS * (KF + KS) == TOTCH and KF % NBUF == 0 and KS % NBUF == 0
IDXPAD = TOTCH + KF

BLK = 1000


_sc_mesh = plsc.VectorSubcoreMesh(
    core_axis_name="c", subcore_axis_name="s", num_cores=NC, num_subcores=NS
)

DW = D // 2


@functools.partial(
    pl.kernel,
    out_type=jax.ShapeDtypeStruct((NP, DW), jnp.int32),
    mesh=_sc_mesh,
    scratch_types=[
        pltpu.VMEM((KF, CS), jnp.int32),
        pltpu.VMEM((NBUF, CS, DW), jnp.int32),
        pltpu.VMEM((KF * C, DW), jnp.int32),
    ]
    + [pltpu.SemaphoreType.DMA] * NBUF,
    compiler_params=pltpu.CompilerParams(
        use_tc_tiling_on_sc=False, needs_layout_passes=False
    ),
)
def _sc_gather_max(h_hbm, idx_hbm, out_hbm, idx_v, rows_v, out_v, *sems):
    core = lax.axis_index("c")
    sub = lax.axis_index("s")
    on_fast = core == 0
    start = lax.select(on_fast, sub * KF, NS * KF + sub * KS)
    kc = lax.select(on_fast, KF, KS)

    @pl.when(kc > 0)
    def _():
        pltpu.sync_copy(idx_hbm.at[pl.ds(start, KF)], idx_v)
        for b in range(NBUF):
            pltpu.async_copy(h_hbm.at[idx_v.at[b]], rows_v.at[b], sems[b])

    nvec = DW // LANES

    def jbody(j, carry):
        for b in range(NBUF):
            chunk = j * NBUF + b
            pltpu.make_async_copy(
                h_hbm.at[idx_v.at[0]], rows_v.at[b], sems[b]
            ).wait()
            rv = rows_v.at[b]
            for c in range(C):
                accs = [
                    plsc.bitcast(rv[c * S, pl.ds(d * LANES, LANES)], jnp.bfloat16)
                    for d in range(nvec)
                ]
                for s in range(1, S):
                    r = c * S + s
                    accs = [
                        jnp.maximum(
                            a,
                            plsc.bitcast(
                                rv[r, pl.ds(d * LANES, LANES)], jnp.bfloat16
                            ),
                        )
                        for d, a in enumerate(accs)
                    ]
                node = chunk * C + c
                for d in range(nvec):
                    out_v[node, pl.ds(d * LANES, LANES)] = plsc.bitcast(
                        accs[d], jnp.int32
                    )

            @pl.when(chunk + NBUF < kc)
            def _():
                pltpu.async_copy(
                    h_hbm.at[idx_v.at[chunk + NBUF]], rows_v.at[b], sems[b]
                )

        return carry

    lax.fori_loop(0, kc // NBUF, jbody, 0)

    @pl.when(on_fast)
    def _():
        pltpu.sync_copy(out_v, out_hbm.at[pl.ds(start * C, KF * C)])

    if KS > 0:

        @pl.when(jnp.logical_not(on_fast))
        def _():
            pltpu.sync_copy(
                out_v.at[pl.ds(0, KS * C)], out_hbm.at[pl.ds(start * C, KS * C)]
            )


def _pack_pairs(y):
    yb = y.astype(jnp.bfloat16)
    lo = lax.bitcast_convert_type(yb[:, :DW], jnp.uint16).astype(jnp.int32)
    hi = lax.bitcast_convert_type(yb[:, DW:], jnp.uint16).astype(jnp.int32)
    return lo | (hi << 16)


def _unpack_pairs(w):
    al = lax.bitcast_convert_type(w << 16, jnp.float32)
    ah = lax.bitcast_convert_type(w & jnp.int32(-65536), jnp.float32)
    return al, ah


def _k_mm_relu(x_ref, w_ref, b_ref, o_ref):
    o_ref[...] = _pack_pairs(
        jnp.maximum(
            jnp.dot(x_ref[...], w_ref[...], preferred_element_type=jnp.float32)
            + b_ref[...],
            0.0,
        )
    )


def _mm_relu(x, w, b):
    return pl.pallas_call(
        _k_mm_relu,
        grid=(N // BLK,),
        in_specs=[
            pl.BlockSpec((BLK, D), lambda i: (i, 0)),
            pl.BlockSpec((D, D), lambda i: (0, 0)),
            pl.BlockSpec((1, D), lambda i: (0, 0)),
        ],
        out_specs=pl.BlockSpec((BLK, DW), lambda i: (i, 0)),
        out_shape=jax.ShapeDtypeStruct((N, DW), jnp.int32),
    )(x, w, b.reshape(1, D))


def _k_fc_sums(x_ref, a_ref, wa_ref, wb_ref, b_ref, y_ref, s_ref):
    y = jnp.dot(x_ref[...], wa_ref[...], preferred_element_type=jnp.float32)
    al, ah = _unpack_pairs(a_ref[...])
    wb = wb_ref[...]
    y = y + jnp.dot(al, wb[:DW], preferred_element_type=jnp.float32)
    y = y + jnp.dot(ah, wb[DW:], preferred_element_type=jnp.float32)
    y = jnp.maximum(y + b_ref[...], 0.0)
    y_ref[...] = y

    @pl.when(pl.program_id(0) == 0)
    def _():
        s_ref[...] = jnp.zeros_like(s_ref)

    s_ref[...] += jnp.concatenate(
        [
            jnp.sum(y, axis=0, keepdims=True),
            jnp.sum(y * y, axis=0, keepdims=True),
            jnp.zeros((6, D), jnp.float32),
        ],
        axis=0,
    )


def _fc_sums(x, a, wa, wb, b):
    return pl.pallas_call(
        _k_fc_sums,
        grid=(N // BLK,),
        in_specs=[
            pl.BlockSpec((BLK, D), lambda i: (i, 0)),
            pl.BlockSpec((BLK, DW), lambda i: (i, 0)),
            pl.BlockSpec((D, D), lambda i: (0, 0)),
            pl.BlockSpec((D, D), lambda i: (0, 0)),
            pl.BlockSpec((1, D), lambda i: (0, 0)),
        ],
        out_specs=[
            pl.BlockSpec((BLK, D), lambda i: (i, 0)),
            pl.BlockSpec((8, D), lambda i: (0, 0)),
        ],
        out_shape=[
            jax.ShapeDtypeStruct((N, D), jnp.float32),
            jax.ShapeDtypeStruct((8, D), jnp.float32),
        ],
    )(x, a, wa, wb, b.reshape(1, D))


def _k_bn_norm_mm(y_ref, s_ref, g_ref, bt_ref, w2_ref, b2_ref, o1_ref, h2_ref):
    s = s_ref[...]
    mean = s[0:1, :] * (1.0 / N)
    var = s[1:2, :] * (1.0 / N) - mean * mean
    inv = lax.rsqrt(var + 1e-5)
    o = g_ref[...] * (y_ref[...] - mean) * inv + bt_ref[...]
    nrm = jnp.sqrt(jnp.sum(o * o, axis=1, keepdims=True))
    o = o / (nrm + 1e-6)
    o1_ref[...] = o
    h2_ref[...] = _pack_pairs(
        jnp.maximum(
            jnp.dot(o, w2_ref[...], preferred_element_type=jnp.float32)
            + b2_ref[...],
            0.0,
        )
    )


def _bn_norm_mm(y, sums, g, bt, w2, b2):
    return pl.pallas_call(
        _k_bn_norm_mm,
        grid=(N // BLK,),
        in_specs=[
            pl.BlockSpec((BLK, D), lambda i: (i, 0)),
            pl.BlockSpec((8, D), lambda i: (0, 0)),
            pl.BlockSpec((1, D), lambda i: (0, 0)),
            pl.BlockSpec((1, D), lambda i: (0, 0)),
            pl.BlockSpec((D, D), lambda i: (0, 0)),
            pl.BlockSpec((1, D), lambda i: (0, 0)),
        ],
        out_specs=[
            pl.BlockSpec((BLK, D), lambda i: (i, 0)),
            pl.BlockSpec((BLK, DW), lambda i: (i, 0)),
        ],
        out_shape=[
            jax.ShapeDtypeStruct((N, D), jnp.float32),
            jax.ShapeDtypeStruct((N, DW), jnp.int32),
        ],
    )(y, sums, g.reshape(1, D), bt.reshape(1, D), w2, b2.reshape(1, D))


def _k_fc_out(x_ref, a_ref, wa_ref, wb_ref, b_ref, o_ref):
    o = jnp.dot(x_ref[...], wa_ref[...], preferred_element_type=jnp.float32)
    al, ah = _unpack_pairs(a_ref[...])
    wb = wb_ref[...]
    o = o + jnp.dot(al, wb[:DW], preferred_element_type=jnp.float32)
    o = o + jnp.dot(ah, wb[DW:], preferred_element_type=jnp.float32)
    o_ref[...] = o + b_ref[...]


def _fc_out(x, a, wa, wb, b):
    return pl.pallas_call(
        _k_fc_out,
        grid=(N // BLK,),
        in_specs=[
            pl.BlockSpec((BLK, D), lambda i: (i, 0)),
            pl.BlockSpec((BLK, DW), lambda i: (i, 0)),
            pl.BlockSpec((D, D), lambda i: (0, 0)),
            pl.BlockSpec((D, D), lambda i: (0, 0)),
            pl.BlockSpec((1, D), lambda i: (0, 0)),
        ],
        out_specs=pl.BlockSpec((BLK, D), lambda i: (i, 0)),
        out_shape=jax.ShapeDtypeStruct((N, D), jnp.float32),
    )(x, a, wa, wb, b.reshape(1, D))


def kernel(features, neigh, Wpool1, bpool1, Wfc1, bfc1, gamma1, beta1,
           Wpool2, bpool2, Wfc2, bfc2):
    x = features.astype(jnp.float32)
    idx = jnp.concatenate(
        [neigh.astype(jnp.int32), jnp.zeros((IDXPAD * C - N, S), jnp.int32)],
        axis=0,
    ).reshape(IDXPAD, CS)

    h1 = _mm_relu(x, Wpool1, bpool1)
    agg1 = _sc_gather_max(h1, idx)[:N]
    y, sums = _fc_sums(x, agg1, Wfc1[:D], Wfc1[D:], bfc1)
    out1, h2 = _bn_norm_mm(y, sums, gamma1, beta1, Wpool2, bpool2)
    agg2 = _sc_gather_max(h2, idx)[:N]
    return _fc_out(out1, agg2, Wfc2[:D], Wfc2[D:], bfc2)
